# baseline (speedup 1.0000x reference)
import functools

import jax
import jax.numpy as jnp
from jax.experimental import pallas as pl
from jax.experimental.pallas import tpu as pltpu

N = 8192
TI = 256
TJ = 2048
NI = N // TI
NJ = N // TJ


def _assd_kernel(pred_ref, realT_ref, out_ref, rowmin_s, accrow_s, colmin_s):
    i = pl.program_id(0)
    j = pl.program_id(1)

    p = pred_ref[...]
    rT = realT_ref[...]

    p2 = jnp.sum(p * p, axis=1, keepdims=True)
    r2 = jnp.sum(rT * rT, axis=0, keepdims=True)
    cross = jax.lax.dot_general(
        p, rT, (((1,), (0,)), ((), ())),
        preferred_element_type=jnp.float32)
    d2 = p2 + r2 - 2.0 * cross
    d2 = jnp.maximum(d2, 1e-12)

    tile_rowmin = jnp.min(d2, axis=1, keepdims=True)
    tile_colmin = jnp.min(d2, axis=0, keepdims=True)

    @pl.when(j == 0)
    def _():
        rowmin_s[...] = tile_rowmin

    @pl.when(j > 0)
    def _():
        rowmin_s[...] = jnp.minimum(rowmin_s[...], tile_rowmin)

    @pl.when(i == 0)
    def _():
        colmin_s[:, pl.ds(j * TJ, TJ)] = tile_colmin

    @pl.when(i > 0)
    def _():
        colmin_s[:, pl.ds(j * TJ, TJ)] = jnp.minimum(
            colmin_s[:, pl.ds(j * TJ, TJ)], tile_colmin)

    @pl.when(j == NJ - 1)
    def _():
        sq = jnp.sqrt(rowmin_s[...])
        @pl.when(i == 0)
        def _():
            accrow_s[...] = sq
        @pl.when(i > 0)
        def _():
            accrow_s[...] = accrow_s[...] + sq

    @pl.when(jnp.logical_and(i == NI - 1, j == NJ - 1))
    def _():
        total_row = jnp.sum(accrow_s[...], keepdims=True)
        total_col = jnp.sum(jnp.sqrt(colmin_s[...]), keepdims=True)
        out_ref[...] = (total_row + total_col) / (2.0 * N)


def kernel(real_pts, pred_pts):
    realT = real_pts.T
    out = pl.pallas_call(
        _assd_kernel,
        grid=(NI, NJ),
        in_specs=[
            pl.BlockSpec((TI, 3), lambda i, j: (i, 0)),
            pl.BlockSpec((3, TJ), lambda i, j: (0, j)),
        ],
        out_specs=pl.BlockSpec((1, 1), lambda i, j: (0, 0)),
        out_shape=jax.ShapeDtypeStruct((1, 1), jnp.float32),
        scratch_shapes=[
            pltpu.VMEM((TI, 1), jnp.float32),
            pltpu.VMEM((TI, 1), jnp.float32),
            pltpu.VMEM((1, N), jnp.float32),
        ],
    )(pred_pts, realT)
    return out[0, 0]


# speedup vs baseline: 2.3071x; 2.3071x over previous
import jax
import jax.numpy as jnp
from jax.experimental import pallas as pl
from jax.experimental.pallas import tpu as pltpu

N = 8192
TI = 512
NI = N // TI


def _assd_kernel(pred_ref, realT_ref, out_ref, accrow_s, colmin_s):
    i = pl.program_id(0)

    p = pred_ref[...]
    rT = realT_ref[...]

    p2 = jnp.sum(p * p, axis=1, keepdims=True)
    r2 = jnp.sum(rT * rT, axis=0, keepdims=True)
    ones_p = jnp.ones((TI, 1), jnp.float32)
    ones_r = jnp.ones((1, N), jnp.float32)
    paug = jnp.concatenate([-2.0 * p, ones_p, p2], axis=1)
    raug = jnp.concatenate([rT, r2, ones_r], axis=0)

    d2 = jax.lax.dot_general(
        paug, raug, (((1,), (0,)), ((), ())),
        preferred_element_type=jnp.float32)

    tile_rowmin = jnp.min(d2, axis=1, keepdims=True)
    tile_colmin = jnp.min(d2, axis=0, keepdims=True)
    row_nn = jnp.sqrt(jnp.maximum(tile_rowmin, 1e-12))

    @pl.when(i == 0)
    def _():
        accrow_s[...] = row_nn
        colmin_s[...] = tile_colmin

    @pl.when(i > 0)
    def _():
        accrow_s[...] = accrow_s[...] + row_nn
        colmin_s[...] = jnp.minimum(colmin_s[...], tile_colmin)

    @pl.when(i == NI - 1)
    def _():
        col_nn = jnp.sqrt(jnp.maximum(colmin_s[...], 1e-12))
        total_row = jnp.sum(accrow_s[...], keepdims=True)
        total_col = jnp.sum(col_nn, keepdims=True)
        out_ref[...] = (total_row + total_col) / (2.0 * N)


def kernel(real_pts, pred_pts):
    realT = real_pts.T
    out = pl.pallas_call(
        _assd_kernel,
        grid=(NI,),
        in_specs=[
            pl.BlockSpec((TI, 3), lambda i: (i, 0)),
            pl.BlockSpec((3, N), lambda i: (0, 0)),
        ],
        out_specs=pl.BlockSpec((1, 1), lambda i: (0, 0)),
        out_shape=jax.ShapeDtypeStruct((1, 1), jnp.float32),
        scratch_shapes=[
            pltpu.VMEM((TI, 1), jnp.float32),
            pltpu.VMEM((1, N), jnp.float32),
        ],
    )(pred_pts, realT)
    return out[0, 0]
